# Initial kernel scaffold; baseline (speedup 1.0000x reference)
import jax
import jax.numpy as jnp
from jax.experimental import pallas as pl


def kernel(logits):
    raise NotImplementedError("write your pallas kernel here")


# speedup vs baseline: 11.4192x; 11.4192x over previous
import jax
import jax.numpy as jnp
from jax import lax
from jax.experimental import pallas as pl
from jax.experimental.pallas import tpu as pltpu
from jax.experimental.pallas import tpu_sc as plsc

MARGIN = 0.2
K = 8
N = 4096
NW = 32
ROWS_PER_W = N // NW
CHUNK = 8
NCHUNK = ROWS_PER_W // CHUNK
NVREG = N // 16


def _sort16(x, descending=False):
    if descending:
        return -lax.sort(-x)
    return lax.sort(x)


def _merge(a_asc, b_desc):
    return jnp.maximum(a_asc, b_desc)


def _body(logits_hbm, out_hbm, buf, accv):
    f32 = jnp.float32
    i32 = jnp.int32
    cid = lax.axis_index("c")
    sid = lax.axis_index("s")
    wid = sid * 2 + cid
    base = wid * ROWS_PER_W

    lane = lax.iota(i32, 16)
    lane0 = lane == 0
    topmask = lane < K
    neg_inf = jnp.full((16,), -jnp.inf, f32)

    def row_body(r, acc):
            i = base + r
            pltpu.sync_copy(logits_hbm.at[i], buf)
            i_vec = jnp.full((16,), i, i32)
            p_vec = plsc.load_gather(buf, [i_vec])
            plsc.store_scatter(buf, [i_vec], neg_inf, mask=lane0)

            def j_body(j, ms):
                v = buf[pl.ds(j * 16, 16)]
                ms = list(ms)
                c = v
                for k in range(K):
                    hi = jnp.maximum(ms[k], c)
                    if k < K - 1:
                        c = jnp.minimum(ms[k], c)
                    ms[k] = hi
                return tuple(ms)

            ms = lax.fori_loop(0, NVREG, j_body, (neg_inf,) * K, unroll=4)

            s = [_sort16(ms[k], descending=bool(k % 2)) for k in range(K)]
            n0 = _sort16(_merge(s[0], s[1]))
            n1 = _sort16(_merge(s[2], s[3]), descending=True)
            n2 = _sort16(_merge(s[4], s[5]))
            n3 = _sort16(_merge(s[6], s[7]), descending=True)
            p0 = _sort16(_merge(n0, n1))
            p1 = _sort16(_merge(n2, n3), descending=True)
            f = _sort16(_merge(p0, p1), descending=True)

            vals = jnp.maximum(f - p_vec + MARGIN, 0.0)
            return acc + jnp.where(topmask, vals, 0.0)

    acc = lax.fori_loop(0, ROWS_PER_W, row_body, jnp.zeros((16,), f32))
    accv[...] = acc
    pltpu.sync_copy(accv, out_hbm.at[wid])


def kernel(logits):
    mesh = plsc.VectorSubcoreMesh(core_axis_name="c", subcore_axis_name="s")
    out = pl.kernel(
        _body,
        out_type=jax.ShapeDtypeStruct((NW, 16), jnp.float32),
        mesh=mesh,
        scratch_types=[
            pltpu.VMEM((N,), jnp.float32),
            pltpu.VMEM((16,), jnp.float32),
        ],
        compiler_params=pltpu.CompilerParams(needs_layout_passes=False),
    )(logits)
    return jnp.sum(out) / N


# speedup vs baseline: 19.8879x; 1.7416x over previous
import jax
import jax.numpy as jnp
from jax import lax
from jax.experimental import pallas as pl
from jax.experimental.pallas import tpu as pltpu
from jax.experimental.pallas import tpu_sc as plsc

MARGIN = 0.2
K = 8
N = 4096
NW = 32
ROWS_PER_W = N // NW
NVREG = N // 16


def _sort16(x, descending=False):
    if descending:
        return -lax.sort(-x)
    return lax.sort(x)


def _merge(a_asc, b_desc):
    return jnp.maximum(a_asc, b_desc)


def _body(logits_hbm, out_hbm, buf_a, buf_b, accv, sem_a, sem_b):
    f32 = jnp.float32
    i32 = jnp.int32
    cid = lax.axis_index("c")
    sid = lax.axis_index("s")
    wid = sid * 2 + cid
    base = wid * ROWS_PER_W

    lane = lax.iota(i32, 16)
    lane0 = lane == 0
    topmask = lane < K
    neg_inf = jnp.full((16,), -jnp.inf, f32)

    def row_topk(i, buf, acc):
        i_vec = jnp.full((16,), i, i32)
        p_vec = plsc.load_gather(buf, [i_vec])
        plsc.store_scatter(buf, [i_vec], neg_inf, mask=lane0)

        def j_body(j, ms):
            v = buf[pl.ds(j * 16, 16)]
            ms = list(ms)
            c = v
            for k in range(K):
                hi = jnp.maximum(ms[k], c)
                if k < K - 1:
                    c = jnp.minimum(ms[k], c)
                ms[k] = hi
            return tuple(ms)

        ms = lax.fori_loop(0, NVREG, j_body, (neg_inf,) * K, unroll=8)

        s = [_sort16(ms[k], descending=bool(k % 2)) for k in range(K)]
        n0 = _sort16(_merge(s[0], s[1]))
        n1 = _sort16(_merge(s[2], s[3]), descending=True)
        n2 = _sort16(_merge(s[4], s[5]))
        n3 = _sort16(_merge(s[6], s[7]), descending=True)
        p0 = _sort16(_merge(n0, n1))
        p1 = _sort16(_merge(n2, n3), descending=True)
        f = _sort16(_merge(p0, p1), descending=True)

        vals = jnp.maximum(f - p_vec + MARGIN, 0.0)
        return acc + jnp.where(topmask, vals, 0.0)

    pltpu.async_copy(logits_hbm.at[base], buf_a, sem_a)

    def pair_body(rr, acc):
        r0 = 2 * rr
        r1 = r0 + 1
        i0 = base + r0
        i1 = base + r1
        pltpu.make_async_copy(logits_hbm.at[i0], buf_a, sem_a).wait()
        pltpu.async_copy(logits_hbm.at[i1], buf_b, sem_b)
        acc = row_topk(i0, buf_a, acc)
        pltpu.make_async_copy(logits_hbm.at[i1], buf_b, sem_b).wait()
        nxt = jnp.minimum(i1 + 1, base + ROWS_PER_W - 1)
        pltpu.async_copy(logits_hbm.at[nxt], buf_a, sem_a)
        acc = row_topk(i1, buf_b, acc)
        return acc

    acc = lax.fori_loop(0, ROWS_PER_W // 2, pair_body, jnp.zeros((16,), f32))
    pltpu.make_async_copy(logits_hbm.at[base], buf_a, sem_a).wait()

    accv[...] = acc
    pltpu.sync_copy(accv, out_hbm.at[wid])


def kernel(logits):
    mesh = plsc.VectorSubcoreMesh(core_axis_name="c", subcore_axis_name="s")
    out = pl.kernel(
        _body,
        out_type=jax.ShapeDtypeStruct((NW, 16), jnp.float32),
        mesh=mesh,
        scratch_types=[
            pltpu.VMEM((N,), jnp.float32),
            pltpu.VMEM((N,), jnp.float32),
            pltpu.VMEM((16,), jnp.float32),
            pltpu.SemaphoreType.DMA,
            pltpu.SemaphoreType.DMA,
        ],
        compiler_params=pltpu.CompilerParams(needs_layout_passes=False),
    )(logits)
    return jnp.sum(out) / N
